# baseline (speedup 1.0000x reference)
import functools
import jax
import jax.numpy as jnp
from jax import lax
from jax.experimental import pallas as pl
from jax.experimental.pallas import tpu as pltpu
from jax.experimental.pallas import tpu_sc as plsc

N_NODES = 10000
D = 128
N_EDGES = 320000
NC = 2
NS = 16
B = 128
NBUF = 2
NB_FAST = 160
NB_SLOW = 2
SB = 56
NB_TOTAL = NS * NB_FAST
E_PAD = NB_TOTAL * B
ACC_ROWS = 10240
TRASH = N_NODES
ROWS_PER_TILE = ACC_ROWS // NS
OUT_CHUNKS = ROWS_PER_TILE // B


def _sc_aggregate():
    mesh = plsc.VectorSubcoreMesh(core_axis_name="c", subcore_axis_name="s")

    @functools.partial(
        pl.kernel,
        mesh=mesh,
        out_type=jax.ShapeDtypeStruct((NC, ACC_ROWS, D), jnp.float32),
        scratch_types=[
            pltpu.VMEM((SB, B), jnp.int32),
            pltpu.VMEM((SB, B), jnp.int32),
            pltpu.VMEM((NBUF, B, D), jnp.float32),
            pltpu.VMEM_SHARED((ACC_ROWS, D), jnp.float32),
            pltpu.SemaphoreType.DMA,
            pltpu.SemaphoreType.DMA,
        ],
    )
    def agg(feat_hbm, src_f, dst_f, src_s, dst_s, out_hbm,
            src_v, dst_v, rows_v, acc, sem0, sem1):
        c = lax.axis_index("c")
        s = lax.axis_index("s")
        sems = (sem0, sem1)

        def gather_start(it, bf):
            pltpu.make_async_copy(
                feat_hbm.at[src_v.at[it]], rows_v.at[bf], sems[bf]).start()

        def run_stage(src_row, dst_row, off, nb):
            pltpu.sync_copy(src_row.at[pl.ds(off, nb)], src_v.at[pl.ds(0, nb)])
            pltpu.sync_copy(dst_row.at[pl.ds(off, nb)], dst_v.at[pl.ds(0, nb)])
            for bf in range(NBUF):
                gather_start(bf, bf)

            def body(g, carry):
                for bf in range(NBUF):
                    it = g * NBUF + bf
                    pltpu.make_async_copy(
                        feat_hbm.at[src_v.at[it]], rows_v.at[bf], sems[bf]).wait()
                    pltpu.sync_copy(rows_v.at[bf], acc.at[dst_v.at[it]], add=True)

                    @pl.when(it + NBUF < nb)
                    def _():
                        gather_start(it + NBUF, bf)
                return carry

            lax.fori_loop(0, nb // NBUF, body, 0)

        @pl.when(c == 0)
        def _():
            zrow = rows_v.at[0]
            nvec = D // 16

            def zstore(i, carry):
                zrow[i // nvec, pl.ds((i % nvec) * 16, 16)] = jnp.zeros(
                    (16,), jnp.float32)
                return carry

            lax.fori_loop(0, B * nvec, zstore, 0)
            for k in range(OUT_CHUNKS):
                pltpu.sync_copy(zrow, acc.at[pl.ds(s * ROWS_PER_TILE + k * B, B)])

        plsc.subcore_barrier()

        @pl.when(c == 0)
        def _():
            for off, nb in ((0, 56), (56, 56), (112, 48)):
                run_stage(src_f.at[s], dst_f.at[s], off, nb)

        @pl.when(c == 1)
        def _():
            run_stage(src_s.at[s], dst_s.at[s], 0, NB_SLOW)

        plsc.subcore_barrier()

        @pl.when(c == 0)
        def _():
            for k in range(OUT_CHUNKS):
                r = s * ROWS_PER_TILE + k * B
                pltpu.sync_copy(acc.at[pl.ds(r, B)], rows_v.at[0])
                pltpu.sync_copy(rows_v.at[0], out_hbm.at[c, pl.ds(r, B)])

    return agg


_AGG = _sc_aggregate()


def _combine_body(p_ref, o_ref):
    o_ref[...] = p_ref[0]


def kernel(feat, edge_index, W, b):
    src = edge_index[0].astype(jnp.int32)
    dst = edge_index[1].astype(jnp.int32)
    pad = E_PAD - N_EDGES
    src_p = jnp.concatenate([src, jnp.zeros((pad,), jnp.int32)]).reshape(NB_TOTAL, B)
    dst_p = jnp.concatenate([dst, jnp.full((pad,), TRASH, jnp.int32)]).reshape(NB_TOTAL, B)
    src_fast = src_p.reshape(NS, NB_FAST, B)
    dst_fast = dst_p.reshape(NS, NB_FAST, B)
    src_slow = jnp.zeros((NS, NB_SLOW, B), jnp.int32)
    dst_slow = jnp.full((NS, NB_SLOW, B), TRASH, jnp.int32)
    partial = _AGG(feat, src_fast, dst_fast, src_slow, dst_slow)
    rows_blk = 2000
    out = pl.pallas_call(
        _combine_body,
        grid=(N_NODES // rows_blk,),
        in_specs=[pl.BlockSpec((1, rows_blk, D), lambda i: (0, i, 0))],
        out_specs=pl.BlockSpec((rows_blk, D), lambda i: (i, 0)),
        out_shape=jax.ShapeDtypeStruct((N_NODES, D), jnp.float32),
    )(partial)
    return out


# speedup vs baseline: 1.9086x; 1.9086x over previous
import functools

import jax
import jax.numpy as jnp
from jax import lax
from jax.experimental import pallas as pl
from jax.experimental.pallas import tpu as pltpu
from jax.experimental.pallas import tpu_sc as plsc

N_NODES = 10000
D = 128
N_EDGES = 320000

NC = 2
NS = 16

B = 128
NBUF = 2
NB_FAST = 152
NB_SLOW = 8
SB = 56
NB_TOTAL = NS * (NB_FAST + NB_SLOW)
E_PAD = NB_TOTAL * B

ACC_ROWS = 10240
TRASH = N_NODES
ROWS_PER_TILE = ACC_ROWS // NS
OUT_CHUNKS = ROWS_PER_TILE // B


def _sc_aggregate():
    mesh = plsc.VectorSubcoreMesh(core_axis_name="c", subcore_axis_name="s")

    @functools.partial(
        pl.kernel,
        mesh=mesh,
        out_type=jax.ShapeDtypeStruct((NC, ACC_ROWS, D), jnp.float32),
        scratch_types=[
            pltpu.VMEM((SB, B), jnp.int32),
            pltpu.VMEM((SB, B), jnp.int32),
            pltpu.VMEM((NBUF, B, D), jnp.float32),
            pltpu.VMEM_SHARED((ACC_ROWS, D), jnp.float32),
            pltpu.SemaphoreType.DMA,
            pltpu.SemaphoreType.DMA,
        ],
    )
    def agg(feat_hbm, src_f, dst_f, src_s, dst_s, out_hbm,
            src_v, dst_v, rows_v, acc, sem0, sem1):
        c = lax.axis_index("c")
        s = lax.axis_index("s")
        sems = (sem0, sem1)

        zrow = rows_v.at[0]
        nvec = D // 16

        def zstore(i, carry):
            zrow[i // nvec, pl.ds((i % nvec) * 16, 16)] = jnp.zeros(
                (16,), jnp.float32)
            return carry

        lax.fori_loop(0, B * nvec, zstore, 0)
        for k in range(OUT_CHUNKS):
            pltpu.sync_copy(
                zrow, acc.at[pl.ds(s * ROWS_PER_TILE + k * B, B)])
        plsc.subcore_barrier()

        def gather_start(it, bf):
            pltpu.make_async_copy(
                feat_hbm.at[src_v.at[it]], rows_v.at[bf], sems[bf]).start()

        def run_stage(src_row, dst_row, off, nb):
            pltpu.sync_copy(
                src_row.at[pl.ds(off, nb)], src_v.at[pl.ds(0, nb)])
            pltpu.sync_copy(
                dst_row.at[pl.ds(off, nb)], dst_v.at[pl.ds(0, nb)])
            for bf in range(NBUF):
                gather_start(bf, bf)

            def body(g, carry):
                for bf in range(NBUF):
                    it = g * NBUF + bf
                    pltpu.make_async_copy(
                        feat_hbm.at[src_v.at[it]], rows_v.at[bf],
                        sems[bf]).wait()
                    pltpu.sync_copy(
                        rows_v.at[bf], acc.at[dst_v.at[it]], add=True)

                    @pl.when(it + NBUF < nb)
                    def _():
                        gather_start(it + NBUF, bf)
                return carry

            lax.fori_loop(0, nb // NBUF, body, 0)

        @pl.when(c == 0)
        def _():
            for off, nb in ((0, 48), (48, 48), (96, 56)):
                run_stage(src_f.at[s], dst_f.at[s], off, nb)

        @pl.when(c == 1)
        def _():
            run_stage(src_s.at[s], dst_s.at[s], 0, NB_SLOW)

        plsc.subcore_barrier()

        for k in range(OUT_CHUNKS):
            r = s * ROWS_PER_TILE + k * B
            pltpu.sync_copy(acc.at[pl.ds(r, B)], rows_v.at[0])
            pltpu.sync_copy(rows_v.at[0], out_hbm.at[c, pl.ds(r, B)])

    return agg


_AGG = _sc_aggregate()


def _combine_body(p_ref, o_ref):
    o_ref[...] = p_ref[0] + p_ref[1]


def kernel(feat, edge_index, W, b):
    src = edge_index[0].astype(jnp.int32)
    dst = edge_index[1].astype(jnp.int32)
    pad = E_PAD - N_EDGES
    src_p = jnp.concatenate(
        [src, jnp.zeros((pad,), jnp.int32)]).reshape(NB_TOTAL, B)
    dst_p = jnp.concatenate(
        [dst, jnp.full((pad,), TRASH, jnp.int32)]).reshape(NB_TOTAL, B)
    nf = NS * NB_FAST
    src_fast = src_p[:nf].reshape(NS, NB_FAST, B)
    dst_fast = dst_p[:nf].reshape(NS, NB_FAST, B)
    src_slow = src_p[nf:].reshape(NS, NB_SLOW, B)
    dst_slow = dst_p[nf:].reshape(NS, NB_SLOW, B)
    partial = _AGG(feat, src_fast, dst_fast, src_slow, dst_slow)

    rows_blk = 2000
    out = pl.pallas_call(
        _combine_body,
        grid=(N_NODES // rows_blk,),
        in_specs=[pl.BlockSpec((NC, rows_blk, D), lambda i: (0, i, 0))],
        out_specs=pl.BlockSpec((rows_blk, D), lambda i: (i, 0)),
        out_shape=jax.ShapeDtypeStruct((N_NODES, D), jnp.float32),
    )(partial)
    return out
